# baseline (speedup 1.0000x reference)
import functools

import jax
import jax.numpy as jnp
from jax import lax
from jax.experimental import pallas as pl
from jax.experimental.pallas import tpu as pltpu
from jax.experimental.pallas import tpu_sc as plsc

C = 128
N = 2048
B = 2
BLK = 128
THR4 = 0.08
THR2 = 0.18
THR1 = 0.22
NROWS = B * N
LANES = 16
K = 128
ECAP = N + K
NW = 32


def _prologue_body(xt_ref, gw_ref, gb_ref, xn_ref, adj_ref, u3_ref, vtab_ref):
    xt = xt_ref[0]
    s = jnp.sum(xt * xt, axis=1, keepdims=True)
    xn = xt / jnp.maximum(jnp.sqrt(s), 1e-12)
    xn_ref[0] = xn
    adj_ref[0] = lax.dot_general(xn, xn, (((1,), (1,)), ((), ())),
                                 preferred_element_type=jnp.float32)
    for l in range(3):
        w = gw_ref[l]
        w1 = w[:C]
        w2 = w[C:]
        u = jnp.dot(xn, w1 - w2, preferred_element_type=jnp.float32) + gb_ref[l]
        v = jnp.dot(xn, w2, preferred_element_type=jnp.float32)
        u3_ref[0, :, l, :] = u
        vtab_ref[l, 0] = v


_SC_MESH = plsc.VectorSubcoreMesh(core_axis_name="c", subcore_axis_name="s")


@functools.partial(
    pl.kernel,
    out_type=jax.ShapeDtypeStruct((NROWS * 3 * C,), jnp.float32),
    mesh=_SC_MESH,
    compiler_params=pltpu.CompilerParams(needs_layout_passes=False),
    scratch_types=[
        pltpu.VMEM((N,), jnp.float32),
        pltpu.VMEM((3 * C,), jnp.float32),
        pltpu.VMEM((ECAP,), jnp.int32),
        pltpu.VMEM((ECAP,), jnp.int32),
        pltpu.VMEM((ECAP,), jnp.int32),
        pltpu.VMEM((ECAP,), jnp.float32),
        pltpu.VMEM((ECAP,), jnp.float32),
        pltpu.VMEM((ECAP,), jnp.float32),
        pltpu.VMEM((K,), jnp.int32),
        pltpu.VMEM((K, C), jnp.float32),
        pltpu.VMEM((3 * C,), jnp.float32),
        pltpu.SemaphoreType.DMA,
    ],
)
def _sc_core(adj_hbm, u3_hbm, vtab_hbm, hout_hbm,
             arow, urow, ei0, ei1, ei2, eg0, eg1, eg2, idxstage, vrows, hbuf,
             sem):
    wid = lax.axis_index("s") * 2 + lax.axis_index("c")
    eis = (ei0, ei1, ei2)
    egs = (eg0, eg1, eg2)

    def per_row(t, carry):
        r = t * NW + wid
        i = lax.rem(r, N)
        bbase = r - i
        blk0 = (i // BLK) * BLK
        pltpu.sync_copy(adj_hbm.at[pl.ds(r * N, N)], arow)
        pltpu.sync_copy(u3_hbm.at[pl.ds(r * 3 * C, 3 * C)], urow)

        def scan_chunk(cidx, cnts):
            c0, c1, c2 = cnts
            col0 = cidx * LANES
            a = arow[pl.ds(col0, LANES)]
            inblk = (col0 >= blk0) & (col0 < blk0 + BLK)
            tmin = jnp.where(inblk, THR4, THR2)
            many = plsc.all_reduce_population_count(a > tmin)[0]

            def work(cs):
                d0, d1, d2 = cs
                g = 1.0 / (1.0 + jnp.exp(-a))
                jv = lax.iota(jnp.int32, LANES) + (col0 + bbase)
                inb = jnp.broadcast_to(inblk, (LANES,))
                m0 = (a > THR4) & inb
                m1 = (a > THR2) & jnp.logical_not(inb)
                m2 = a > THR1
                n0 = plsc.all_reduce_population_count(m0)[0]
                n1 = plsc.all_reduce_population_count(m1)[0]
                n2 = plsc.all_reduce_population_count(m2)[0]

                @pl.when(n0 > 0)
                def _():
                    plsc.store_compressed(ei0.at[pl.ds(d0, LANES)], jv, mask=m0)
                    plsc.store_compressed(eg0.at[pl.ds(d0, LANES)], g, mask=m0)

                @pl.when(n1 > 0)
                def _():
                    plsc.store_compressed(ei1.at[pl.ds(d1, LANES)],
                                          jv + NROWS, mask=m1)
                    plsc.store_compressed(eg1.at[pl.ds(d1, LANES)], g, mask=m1)

                @pl.when(n2 > 0)
                def _():
                    plsc.store_compressed(ei2.at[pl.ds(d2, LANES)],
                                          jv + 2 * NROWS, mask=m2)
                    plsc.store_compressed(eg2.at[pl.ds(d2, LANES)], g, mask=m2)

                return (d0 + n0, d1 + n1, d2 + n2)

            return lax.cond(many > 0, work, lambda cs: cs, (c0, c1, c2))

        z = jnp.int32(0)
        cnts = lax.fori_loop(0, N // LANES, scan_chunk, (z, z, z))

        zi = jnp.zeros((LANES,), jnp.int32)
        for l in range(3):
            for tt in range(K // LANES):
                eis[l][pl.ds(cnts[l] + tt * LANES, LANES)] = zi

        for l in range(3):
            ei = eis[l]
            eg = egs[l]
            cn = cnts[l]
            us = tuple(urow[pl.ds(l * C + k * LANES, LANES)] for k in range(8))
            nchunks = (cn + K - 1) // K

            def chunk_loop(q, accs, ei=ei, eg=eg, cn=cn, us=us):
                for tt in range(K // LANES):
                    idxstage[pl.ds(tt * LANES, LANES)] = (
                        ei[pl.ds(q * K + tt * LANES, LANES)])
                pltpu.async_copy(vtab_hbm.at[idxstage], vrows, sem).wait()
                nin = jnp.minimum(K, cn - q * K)

                def edge(e, acc2):
                    g = eg[pl.ds(q * K + e, LANES)][0]
                    return tuple(
                        jnp.maximum(acc2[k],
                                    g * (us[k] + vrows[e, pl.ds(k * LANES,
                                                                LANES)]))
                        for k in range(8))

                return lax.fori_loop(0, nin, edge, accs)

            init = tuple(jnp.full((LANES,), -jnp.inf, jnp.float32)
                         for _ in range(8))
            accs = lax.fori_loop(0, nchunks, chunk_loop, init)
            for k in range(8):
                hbuf[pl.ds(l * C + k * LANES, LANES)] = jnp.where(
                    accs[k] > -1e37, accs[k], 0.0)

        pltpu.sync_copy(hbuf, hout_hbm.at[pl.ds(r * 3 * C, 3 * C)])
        return carry

    lax.fori_loop(0, NROWS // NW, per_row, 0)


def _sigmoid(z):
    return 1.0 / (1.0 + jnp.exp(-z))


def _epi_body(h_ref, xn_ref, bng_ref, bnb_ref, attw_ref, attb_ref,
              w1_ref, b1_ref, w2_ref, b2_ref, law_ref, lab_ref,
              fuw_ref, fub_ref, out_ref):
    xn = xn_ref[...]
    ys, ws, avals = [], [], []
    for l in range(3):
        h = h_ref[:, l * C:(l + 1) * C]
        mu = jnp.mean(h, axis=0, keepdims=True)
        var = jnp.mean((h - mu) ** 2, axis=0, keepdims=True)
        hn = bng_ref[l] * (h - mu) / jnp.sqrt(var + 1e-5) + bnb_ref[l]
        hr = hn + xn
        gate = _sigmoid(jnp.dot(hr, attw_ref[l],
                                preferred_element_type=jnp.float32)
                        + attb_ref[l])
        y = jnp.maximum(gate * hr, 0.0)
        p0 = jnp.max(y[:N], axis=0, keepdims=True)
        p1 = jnp.max(y[N:], axis=0, keepdims=True)
        p = jnp.concatenate([p0, p1], axis=0)
        hid = jnp.maximum(jnp.dot(p, w1_ref[l],
                                  preferred_element_type=jnp.float32)
                          + b1_ref[l], 0.0)
        hcap = jnp.dot(hid, w2_ref[l],
                       preferred_element_type=jnp.float32) + b2_ref[l]
        w = _sigmoid(hcap)
        a = jnp.dot(w, law_ref[l],
                    preferred_element_type=jnp.float32) + lab_ref[l]
        ys.append(y)
        ws.append(w)
        avals.append(a)
    amax = jnp.maximum(jnp.maximum(avals[0], avals[1]), avals[2])
    es = [jnp.exp(a - amax) for a in avals]
    ssum = es[0] + es[1] + es[2]
    for b in range(B):
        fb = jnp.zeros((N, C), jnp.float32)
        for l in range(3):
            scale = (es[l][b:b + 1, :] / ssum[b:b + 1, :]) * ws[l][b:b + 1, :]
            fb = fb + scale * ys[l][b * N:(b + 1) * N]
        ob = jnp.dot(fb, fuw_ref[...],
                     preferred_element_type=jnp.float32) + fub_ref[...]
        out_ref[b] = ob.T


def kernel(x,
           gcn_w_0, gcn_b_0, bn_g_0, bn_b_0, att_w_0, att_b_0,
           gcn_w_1, gcn_b_1, bn_g_1, bn_b_1, att_w_1, att_b_1,
           gcn_w_2, gcn_b_2, bn_g_2, bn_b_2, att_w_2, att_b_2,
           lineA4_w, lineA4_b, mlpCA4_w1, mlpCA4_b1, mlpCA4_w2, mlpCA4_b2,
           lineA2_w, lineA2_b, mlpCA2_w1, mlpCA2_b1, mlpCA2_w2, mlpCA2_b2,
           lineA1_w, lineA1_b, mlpCA1_w1, mlpCA1_b1, mlpCA1_w2, mlpCA1_b2,
           lineFu_w, lineFu_b):
    xt = jnp.transpose(x, (0, 2, 1))
    gw = jnp.stack([gcn_w_0, gcn_w_1, gcn_w_2])
    gb = jnp.stack([gcn_b_0, gcn_b_1, gcn_b_2])[:, None, :]

    xn, adj, u3, vtab = pl.pallas_call(
        _prologue_body,
        grid=(B,),
        in_specs=[
            pl.BlockSpec((1, N, C), lambda b: (b, 0, 0)),
            pl.BlockSpec((3, 2 * C, C), lambda b: (0, 0, 0)),
            pl.BlockSpec((3, 1, C), lambda b: (0, 0, 0)),
        ],
        out_specs=[
            pl.BlockSpec((1, N, C), lambda b: (b, 0, 0)),
            pl.BlockSpec((1, N, N), lambda b: (b, 0, 0)),
            pl.BlockSpec((1, N, 3, C), lambda b: (b, 0, 0, 0)),
            pl.BlockSpec((3, 1, N, C), lambda b: (0, b, 0, 0)),
        ],
        out_shape=[
            jax.ShapeDtypeStruct((B, N, C), jnp.float32),
            jax.ShapeDtypeStruct((B, N, N), jnp.float32),
            jax.ShapeDtypeStruct((B, N, 3, C), jnp.float32),
            jax.ShapeDtypeStruct((3, B, N, C), jnp.float32),
        ],
    )(xt, gw, gb)

    hout = _sc_core(adj.reshape(-1), u3.reshape(-1),
                    vtab.reshape(3 * B * N, C))
    hcat = hout.reshape(NROWS, 3 * C)
    xnf = xn.reshape(NROWS, C)

    bng = jnp.stack([bn_g_0, bn_g_1, bn_g_2])[:, None, :]
    bnb = jnp.stack([bn_b_0, bn_b_1, bn_b_2])[:, None, :]
    attw = jnp.stack([att_w_0, att_w_1, att_w_2])
    attb = jnp.stack([att_b_0, att_b_1, att_b_2])[:, None, :]
    w1s = jnp.stack([mlpCA4_w1, mlpCA2_w1, mlpCA1_w1])
    b1s = jnp.stack([mlpCA4_b1, mlpCA2_b1, mlpCA1_b1])[:, None, :]
    w2s = jnp.stack([mlpCA4_w2, mlpCA2_w2, mlpCA1_w2])
    b2s = jnp.stack([mlpCA4_b2, mlpCA2_b2, mlpCA1_b2])[:, None, :]
    law = jnp.stack([lineA4_w, lineA2_w, lineA1_w])
    lab = jnp.stack([lineA4_b, lineA2_b, lineA1_b])[:, None, :]

    out = pl.pallas_call(
        _epi_body,
        out_shape=jax.ShapeDtypeStruct((B, C, N), jnp.float32),
    )(hcat, xnf, bng, bnb, attw, attb, w1s, b1s, w2s, b2s, law, lab,
      lineFu_w, lineFu_b[None, :])
    return out


# speedup vs baseline: 28.9721x; 28.9721x over previous
import functools

import jax
import jax.numpy as jnp
from jax import lax
from jax.experimental import pallas as pl
from jax.experimental.pallas import tpu as pltpu
from jax.experimental.pallas import tpu_sc as plsc

C = 128
N = 2048
B = 2
BLK = 128
THR4 = 0.08
THR2 = 0.18
THR1 = 0.22
NROWS = B * N
LANES = 16
TJ = 64
NW = 32


def _prologue_body(xt_ref, gw_ref, gb_ref, xn_ref, adj_ref, utab_ref, vtab_ref):
    xt = xt_ref[0]
    s = jnp.sum(xt * xt, axis=1, keepdims=True)
    xn = xt / jnp.maximum(jnp.sqrt(s), 1e-12)
    xn_ref[0] = xn
    adj_ref[0] = lax.dot_general(xn, xn, (((1,), (1,)), ((), ())),
                                 preferred_element_type=jnp.float32)
    for l in range(3):
        w = gw_ref[l]
        w1 = w[:C]
        w2 = w[C:]
        u = jnp.dot(xn, w1 - w2, preferred_element_type=jnp.float32) + gb_ref[l]
        v = jnp.dot(xn, w2, preferred_element_type=jnp.float32)
        utab_ref[l, 0] = u
        vtab_ref[l, 0] = v


_SC_MESH = plsc.VectorSubcoreMesh(core_axis_name="c", subcore_axis_name="s")


@functools.partial(
    pl.kernel,
    out_type=jax.ShapeDtypeStruct((3 * NROWS, C), jnp.float32),
    mesh=_SC_MESH,
    compiler_params=pltpu.CompilerParams(needs_layout_passes=False),
    scratch_types=[
        pltpu.VMEM((BLK, C), jnp.float32),
        pltpu.VMEM((BLK, C), jnp.float32),
        pltpu.VMEM((BLK, C), jnp.float32),
        pltpu.VMEM((BLK, C), jnp.float32),
        pltpu.VMEM((BLK, C), jnp.float32),
        pltpu.VMEM((BLK, C), jnp.float32),
        pltpu.VMEM((TJ, BLK), jnp.float32),
        pltpu.VMEM((TJ, C), jnp.float32),
        pltpu.VMEM((TJ, C), jnp.float32),
        pltpu.VMEM((BLK + LANES,), jnp.int32),
        pltpu.VMEM((BLK + LANES,), jnp.float32),
        pltpu.VMEM((BLK + LANES,), jnp.float32),
        pltpu.SemaphoreType.DMA,
    ],
)
def _sc_core(adj_hbm, utab_hbm, vtab_hbm, hout_hbm,
             u_a, u_b, u2, h0, h1, h2, adjchunk, vbuf1, vbuf2,
             lbuf, gbuf, abuf, sem):
    wid = lax.axis_index("s") * 2 + lax.axis_index("c")
    bbase = (wid // 16) * N
    i0 = (wid % 16) * BLK
    neg = jnp.full((LANES,), -jnp.inf, jnp.float32)

    pltpu.sync_copy(utab_hbm.at[pl.ds(2 * NROWS + bbase + i0, BLK)], u2)
    for href in (h0, h1, h2):
        def initrow(ri, _, href=href):
            for k in range(C // LANES):
                href[ri, pl.ds(k * LANES, LANES)] = neg
            return 0
        lax.fori_loop(0, BLK, initrow, 0)

    def sweep(thr, uref, href, vl_base, chunk0, nchunks, skip_own):
        def do_chunk(ci, _):
            j0 = chunk0 + ci * TJ
            own = (j0 >= i0) & (j0 < i0 + BLK)

            @pl.when(jnp.logical_not(own) if skip_own else (ci >= 0))
            def _():
                pltpu.sync_copy(
                    adj_hbm.at[pl.ds(bbase + j0, TJ), pl.ds(i0, BLK)],
                    adjchunk)
                pltpu.sync_copy(
                    vtab_hbm.at[pl.ds(vl_base + bbase + j0, TJ)], vbuf1)
                pltpu.sync_copy(
                    vtab_hbm.at[pl.ds(2 * NROWS + bbase + j0, TJ)], vbuf2)

                def per_j(jl, _):
                    cnt = jnp.int32(0)
                    for kk in range(BLK // LANES):
                        a = adjchunk[jl, pl.ds(kk * LANES, LANES)]
                        m = a > thr
                        n = plsc.all_reduce_population_count(m)[0]

                        @pl.when(n > 0)
                        def _(cnt=cnt, a=a, m=m, kk=kk):
                            g = 1.0 / (1.0 + jnp.exp(-a))
                            ids = lax.iota(jnp.int32, LANES) + (kk * LANES)
                            plsc.store_compressed(
                                lbuf.at[pl.ds(cnt, LANES)], ids, mask=m)
                            plsc.store_compressed(
                                gbuf.at[pl.ds(cnt, LANES)], g, mask=m)
                            plsc.store_compressed(
                                abuf.at[pl.ds(cnt, LANES)], a, mask=m)
                        cnt = cnt + n

                    def edge(e, _):
                        lane = lbuf[pl.ds(e, LANES)][0]
                        g = gbuf[pl.ds(e, LANES)][0]
                        av = abuf[pl.ds(e, LANES)][0]
                        for k in range(C // LANES):
                            sl = pl.ds(k * LANES, LANES)
                            cand = g * (uref[lane, sl] + vbuf1[jl, sl])
                            href[lane, sl] = jnp.maximum(href[lane, sl], cand)

                        @pl.when(av > THR1)
                        def _():
                            for k in range(C // LANES):
                                sl = pl.ds(k * LANES, LANES)
                                cand2 = g * (u2[lane, sl] + vbuf2[jl, sl])
                                h2[lane, sl] = jnp.maximum(h2[lane, sl], cand2)
                        return 0

                    lax.fori_loop(0, cnt, edge, 0)
                    return 0

                lax.fori_loop(0, TJ, per_j, 0)
            return 0

        lax.fori_loop(0, nchunks, do_chunk, 0)

    pltpu.sync_copy(utab_hbm.at[pl.ds(bbase + i0, BLK)], u_a)
    sweep(THR4, u_a, h0, 0, i0, BLK // TJ, False)
    pltpu.sync_copy(utab_hbm.at[pl.ds(NROWS + bbase + i0, BLK)], u_b)
    sweep(THR2, u_b, h1, NROWS, 0, N // TJ, True)

    for l, href in ((0, h0), (1, h1), (2, h2)):
        def finrow(ri, _, href=href):
            for k in range(C // LANES):
                sl = pl.ds(k * LANES, LANES)
                v = href[ri, sl]
                href[ri, sl] = jnp.where(v > -1e37, v, 0.0)
            return 0
        lax.fori_loop(0, BLK, finrow, 0)
        pltpu.sync_copy(href, hout_hbm.at[pl.ds(l * NROWS + bbase + i0, BLK)])


def _sigmoid(z):
    return 1.0 / (1.0 + jnp.exp(-z))


def _epi_body(h_ref, xn_ref, bng_ref, bnb_ref, attw_ref, attb_ref,
              w1_ref, b1_ref, w2_ref, b2_ref, law_ref, lab_ref,
              fuw_ref, fub_ref, out_ref):
    xn = xn_ref[...]
    ys, ws, avals = [], [], []
    for l in range(3):
        h = h_ref[l]
        mu = jnp.mean(h, axis=0, keepdims=True)
        var = jnp.mean((h - mu) ** 2, axis=0, keepdims=True)
        hn = bng_ref[l] * (h - mu) / jnp.sqrt(var + 1e-5) + bnb_ref[l]
        hr = hn + xn
        gate = _sigmoid(jnp.dot(hr, attw_ref[l],
                                preferred_element_type=jnp.float32)
                        + attb_ref[l])
        y = jnp.maximum(gate * hr, 0.0)
        p0 = jnp.max(y[:N], axis=0, keepdims=True)
        p1 = jnp.max(y[N:], axis=0, keepdims=True)
        p = jnp.concatenate([p0, p1], axis=0)
        hid = jnp.maximum(jnp.dot(p, w1_ref[l],
                                  preferred_element_type=jnp.float32)
                          + b1_ref[l], 0.0)
        hcap = jnp.dot(hid, w2_ref[l],
                       preferred_element_type=jnp.float32) + b2_ref[l]
        w = _sigmoid(hcap)
        a = jnp.dot(w, law_ref[l],
                    preferred_element_type=jnp.float32) + lab_ref[l]
        ys.append(y)
        ws.append(w)
        avals.append(a)
    amax = jnp.maximum(jnp.maximum(avals[0], avals[1]), avals[2])
    es = [jnp.exp(a - amax) for a in avals]
    ssum = es[0] + es[1] + es[2]
    for b in range(B):
        fb = jnp.zeros((N, C), jnp.float32)
        for l in range(3):
            scale = (es[l][b:b + 1, :] / ssum[b:b + 1, :]) * ws[l][b:b + 1, :]
            fb = fb + scale * ys[l][b * N:(b + 1) * N]
        ob = jnp.dot(fb, fuw_ref[...],
                     preferred_element_type=jnp.float32) + fub_ref[...]
        out_ref[b] = ob.T


def kernel(x,
           gcn_w_0, gcn_b_0, bn_g_0, bn_b_0, att_w_0, att_b_0,
           gcn_w_1, gcn_b_1, bn_g_1, bn_b_1, att_w_1, att_b_1,
           gcn_w_2, gcn_b_2, bn_g_2, bn_b_2, att_w_2, att_b_2,
           lineA4_w, lineA4_b, mlpCA4_w1, mlpCA4_b1, mlpCA4_w2, mlpCA4_b2,
           lineA2_w, lineA2_b, mlpCA2_w1, mlpCA2_b1, mlpCA2_w2, mlpCA2_b2,
           lineA1_w, lineA1_b, mlpCA1_w1, mlpCA1_b1, mlpCA1_w2, mlpCA1_b2,
           lineFu_w, lineFu_b):
    xt = jnp.transpose(x, (0, 2, 1))
    gw = jnp.stack([gcn_w_0, gcn_w_1, gcn_w_2])
    gb = jnp.stack([gcn_b_0, gcn_b_1, gcn_b_2])[:, None, :]

    xn, adj, utab, vtab = pl.pallas_call(
        _prologue_body,
        grid=(B,),
        in_specs=[
            pl.BlockSpec((1, N, C), lambda b: (b, 0, 0)),
            pl.BlockSpec((3, 2 * C, C), lambda b: (0, 0, 0)),
            pl.BlockSpec((3, 1, C), lambda b: (0, 0, 0)),
        ],
        out_specs=[
            pl.BlockSpec((1, N, C), lambda b: (b, 0, 0)),
            pl.BlockSpec((1, N, N), lambda b: (b, 0, 0)),
            pl.BlockSpec((3, 1, N, C), lambda b: (0, b, 0, 0)),
            pl.BlockSpec((3, 1, N, C), lambda b: (0, b, 0, 0)),
        ],
        out_shape=[
            jax.ShapeDtypeStruct((B, N, C), jnp.float32),
            jax.ShapeDtypeStruct((B, N, N), jnp.float32),
            jax.ShapeDtypeStruct((3, B, N, C), jnp.float32),
            jax.ShapeDtypeStruct((3, B, N, C), jnp.float32),
        ],
    )(xt, gw, gb)

    hout = _sc_core(adj.reshape(NROWS, N), utab.reshape(3 * NROWS, C),
                    vtab.reshape(3 * NROWS, C))
    h3 = hout.reshape(3, NROWS, C)
    xnf = xn.reshape(NROWS, C)

    bng = jnp.stack([bn_g_0, bn_g_1, bn_g_2])[:, None, :]
    bnb = jnp.stack([bn_b_0, bn_b_1, bn_b_2])[:, None, :]
    attw = jnp.stack([att_w_0, att_w_1, att_w_2])
    attb = jnp.stack([att_b_0, att_b_1, att_b_2])[:, None, :]
    w1s = jnp.stack([mlpCA4_w1, mlpCA2_w1, mlpCA1_w1])
    b1s = jnp.stack([mlpCA4_b1, mlpCA2_b1, mlpCA1_b1])[:, None, :]
    w2s = jnp.stack([mlpCA4_w2, mlpCA2_w2, mlpCA1_w2])
    b2s = jnp.stack([mlpCA4_b2, mlpCA2_b2, mlpCA1_b2])[:, None, :]
    law = jnp.stack([lineA4_w, lineA2_w, lineA1_w])
    lab = jnp.stack([lineA4_b, lineA2_b, lineA1_b])[:, None, :]

    out = pl.pallas_call(
        _epi_body,
        out_shape=jax.ShapeDtypeStruct((B, C, N), jnp.float32),
    )(h3, xnf, bng, bnb, attw, attb, w1s, b1s, w2s, b2s, law, lab,
      lineFu_w, lineFu_b[None, :])
    return out


# speedup vs baseline: 56.5650x; 1.9524x over previous
import functools

import jax
import jax.numpy as jnp
from jax import lax
from jax.experimental import pallas as pl
from jax.experimental.pallas import tpu as pltpu
from jax.experimental.pallas import tpu_sc as plsc

C = 128
N = 2048
B = 2
BLK = 128
THR4 = 0.08
THR2 = 0.18
THR1 = 0.22
NROWS = B * N
LANES = 16
TJ = 64
NW = 32


def _prologue_body(xt_ref, gw_ref, gb_ref, xn_ref, adj_ref, utab_ref, vtab_ref):
    xt = xt_ref[0]
    s = jnp.sum(xt * xt, axis=1, keepdims=True)
    xn = xt / jnp.maximum(jnp.sqrt(s), 1e-12)
    xn_ref[0] = xn
    adj_ref[0] = lax.dot_general(xn, xn, (((1,), (1,)), ((), ())),
                                 preferred_element_type=jnp.float32)
    for l in range(3):
        w = gw_ref[l]
        w1 = w[:C]
        w2 = w[C:]
        u = jnp.dot(xn, w1 - w2, preferred_element_type=jnp.float32) + gb_ref[l]
        v = jnp.dot(xn, w2, preferred_element_type=jnp.float32)
        utab_ref[l, 0] = u
        vtab_ref[l, 0] = v


_SC_MESH = plsc.VectorSubcoreMesh(core_axis_name="c", subcore_axis_name="s")


@functools.partial(
    pl.kernel,
    out_type=jax.ShapeDtypeStruct((3 * NROWS, C), jnp.float32),
    mesh=_SC_MESH,
    compiler_params=pltpu.CompilerParams(needs_layout_passes=False),
    scratch_types=[
        pltpu.VMEM((BLK, C), jnp.float32),
        pltpu.VMEM((BLK, C), jnp.float32),
        pltpu.VMEM((BLK, C), jnp.float32),
        pltpu.VMEM((BLK, C), jnp.float32),
        pltpu.VMEM((BLK, C), jnp.float32),
        pltpu.VMEM((BLK, C), jnp.float32),
        pltpu.VMEM((TJ, BLK), jnp.float32),
        pltpu.VMEM((TJ, C), jnp.float32),
        pltpu.VMEM((TJ, C), jnp.float32),
        pltpu.VMEM((BLK + LANES,), jnp.int32),
        pltpu.VMEM((BLK + LANES,), jnp.float32),
        pltpu.VMEM((BLK + LANES,), jnp.float32),
        pltpu.SemaphoreType.DMA,
    ],
)
def _sc_core(adj_hbm, utab_hbm, vtab_hbm, hout_hbm,
             u_a, u_b, u2, h0, h1, h2, adjchunk, vbuf1, vbuf2,
             lbuf, gbuf, abuf, sem):
    wid = lax.axis_index("s") * 2 + lax.axis_index("c")
    bbase = (wid // 16) * N
    i0 = (wid % 16) * BLK
    neg = jnp.full((LANES,), -jnp.inf, jnp.float32)

    pltpu.sync_copy(utab_hbm.at[pl.ds(2 * NROWS + bbase + i0, BLK)], u2)
    for href in (h0, h1, h2):
        def initrow(ri, _, href=href):
            for k in range(C // LANES):
                href[ri, pl.ds(k * LANES, LANES)] = neg
            return 0
        lax.fori_loop(0, BLK, initrow, 0)

    def sweep(thr, uref, href, vl_base, chunk0, nchunks, skip_own):
        def do_chunk(ci, _):
            j0 = chunk0 + ci * TJ
            own = (j0 >= i0) & (j0 < i0 + BLK)

            @pl.when(jnp.logical_not(own) if skip_own else (ci >= 0))
            def _():
                pltpu.sync_copy(
                    adj_hbm.at[pl.ds(bbase + j0, TJ), pl.ds(i0, BLK)],
                    adjchunk)
                pltpu.sync_copy(
                    vtab_hbm.at[pl.ds(vl_base + bbase + j0, TJ)], vbuf1)
                pltpu.sync_copy(
                    vtab_hbm.at[pl.ds(2 * NROWS + bbase + j0, TJ)], vbuf2)

                def per_j(jl, _):
                    avs = [adjchunk[jl, pl.ds(kk * LANES, LANES)]
                           for kk in range(BLK // LANES)]
                    ms = [a > thr for a in avs]
                    ns = [plsc.all_reduce_population_count(m)[0] for m in ms]
                    offs = []
                    cnt = jnp.int32(0)
                    for n in ns:
                        offs.append(cnt)
                        cnt = cnt + n

                    @pl.when(cnt > 0)
                    def _():
                        for kk in range(BLK // LANES):
                            @pl.when(ns[kk] > 0)
                            def _(kk=kk):
                                a = avs[kk]
                                m = ms[kk]
                                g = 1.0 / (1.0 + jnp.exp(-a))
                                ids = lax.iota(jnp.int32, LANES) + (kk * LANES)
                                plsc.store_compressed(
                                    lbuf.at[pl.ds(offs[kk], LANES)], ids,
                                    mask=m)
                                plsc.store_compressed(
                                    gbuf.at[pl.ds(offs[kk], LANES)], g, mask=m)
                                plsc.store_compressed(
                                    abuf.at[pl.ds(offs[kk], LANES)], a, mask=m)

                        @plsc.parallel_loop(0, cnt, 1, unroll=2)
                        def edge(e):
                            lane = lbuf[pl.ds(e, LANES)][0]
                            g = gbuf[pl.ds(e, LANES)][0]
                            av = abuf[pl.ds(e, LANES)][0]
                            for k in range(C // LANES):
                                sl = pl.ds(k * LANES, LANES)
                                cand = g * (uref[lane, sl] + vbuf1[jl, sl])
                                href[lane, sl] = jnp.maximum(href[lane, sl],
                                                             cand)

                            @pl.when(av > THR1)
                            def _():
                                for k in range(C // LANES):
                                    sl = pl.ds(k * LANES, LANES)
                                    cand2 = g * (u2[lane, sl] + vbuf2[jl, sl])
                                    h2[lane, sl] = jnp.maximum(h2[lane, sl],
                                                               cand2)
                    return 0

                lax.fori_loop(0, TJ, per_j, 0)
            return 0

        lax.fori_loop(0, nchunks, do_chunk, 0)

    pltpu.sync_copy(utab_hbm.at[pl.ds(bbase + i0, BLK)], u_a)
    sweep(THR4, u_a, h0, 0, i0, BLK // TJ, False)
    pltpu.sync_copy(utab_hbm.at[pl.ds(NROWS + bbase + i0, BLK)], u_b)
    sweep(THR2, u_b, h1, NROWS, 0, N // TJ, True)

    for l, href in ((0, h0), (1, h1), (2, h2)):
        def finrow(ri, _, href=href):
            for k in range(C // LANES):
                sl = pl.ds(k * LANES, LANES)
                v = href[ri, sl]
                href[ri, sl] = jnp.where(v > -1e37, v, 0.0)
            return 0
        lax.fori_loop(0, BLK, finrow, 0)
        pltpu.sync_copy(href, hout_hbm.at[pl.ds(l * NROWS + bbase + i0, BLK)])


def _sigmoid(z):
    return 1.0 / (1.0 + jnp.exp(-z))


def _epi_body(h_ref, xn_ref, bng_ref, bnb_ref, attw_ref, attb_ref,
              w1_ref, b1_ref, w2_ref, b2_ref, law_ref, lab_ref,
              fuw_ref, fub_ref, out_ref):
    xn = xn_ref[...]
    ys, ws, avals = [], [], []
    for l in range(3):
        h = h_ref[l]
        mu = jnp.mean(h, axis=0, keepdims=True)
        var = jnp.mean((h - mu) ** 2, axis=0, keepdims=True)
        hn = bng_ref[l] * (h - mu) / jnp.sqrt(var + 1e-5) + bnb_ref[l]
        hr = hn + xn
        gate = _sigmoid(jnp.dot(hr, attw_ref[l],
                                preferred_element_type=jnp.float32)
                        + attb_ref[l])
        y = jnp.maximum(gate * hr, 0.0)
        p0 = jnp.max(y[:N], axis=0, keepdims=True)
        p1 = jnp.max(y[N:], axis=0, keepdims=True)
        p = jnp.concatenate([p0, p1], axis=0)
        hid = jnp.maximum(jnp.dot(p, w1_ref[l],
                                  preferred_element_type=jnp.float32)
                          + b1_ref[l], 0.0)
        hcap = jnp.dot(hid, w2_ref[l],
                       preferred_element_type=jnp.float32) + b2_ref[l]
        w = _sigmoid(hcap)
        a = jnp.dot(w, law_ref[l],
                    preferred_element_type=jnp.float32) + lab_ref[l]
        ys.append(y)
        ws.append(w)
        avals.append(a)
    amax = jnp.maximum(jnp.maximum(avals[0], avals[1]), avals[2])
    es = [jnp.exp(a - amax) for a in avals]
    ssum = es[0] + es[1] + es[2]
    for b in range(B):
        fb = jnp.zeros((N, C), jnp.float32)
        for l in range(3):
            scale = (es[l][b:b + 1, :] / ssum[b:b + 1, :]) * ws[l][b:b + 1, :]
            fb = fb + scale * ys[l][b * N:(b + 1) * N]
        ob = jnp.dot(fb, fuw_ref[...],
                     preferred_element_type=jnp.float32) + fub_ref[...]
        out_ref[b] = ob.T


def kernel(x,
           gcn_w_0, gcn_b_0, bn_g_0, bn_b_0, att_w_0, att_b_0,
           gcn_w_1, gcn_b_1, bn_g_1, bn_b_1, att_w_1, att_b_1,
           gcn_w_2, gcn_b_2, bn_g_2, bn_b_2, att_w_2, att_b_2,
           lineA4_w, lineA4_b, mlpCA4_w1, mlpCA4_b1, mlpCA4_w2, mlpCA4_b2,
           lineA2_w, lineA2_b, mlpCA2_w1, mlpCA2_b1, mlpCA2_w2, mlpCA2_b2,
           lineA1_w, lineA1_b, mlpCA1_w1, mlpCA1_b1, mlpCA1_w2, mlpCA1_b2,
           lineFu_w, lineFu_b):
    xt = jnp.transpose(x, (0, 2, 1))
    gw = jnp.stack([gcn_w_0, gcn_w_1, gcn_w_2])
    gb = jnp.stack([gcn_b_0, gcn_b_1, gcn_b_2])[:, None, :]

    xn, adj, utab, vtab = pl.pallas_call(
        _prologue_body,
        grid=(B,),
        in_specs=[
            pl.BlockSpec((1, N, C), lambda b: (b, 0, 0)),
            pl.BlockSpec((3, 2 * C, C), lambda b: (0, 0, 0)),
            pl.BlockSpec((3, 1, C), lambda b: (0, 0, 0)),
        ],
        out_specs=[
            pl.BlockSpec((1, N, C), lambda b: (b, 0, 0)),
            pl.BlockSpec((1, N, N), lambda b: (b, 0, 0)),
            pl.BlockSpec((3, 1, N, C), lambda b: (0, b, 0, 0)),
            pl.BlockSpec((3, 1, N, C), lambda b: (0, b, 0, 0)),
        ],
        out_shape=[
            jax.ShapeDtypeStruct((B, N, C), jnp.float32),
            jax.ShapeDtypeStruct((B, N, N), jnp.float32),
            jax.ShapeDtypeStruct((3, B, N, C), jnp.float32),
            jax.ShapeDtypeStruct((3, B, N, C), jnp.float32),
        ],
    )(xt, gw, gb)

    hout = _sc_core(adj.reshape(NROWS, N), utab.reshape(3 * NROWS, C),
                    vtab.reshape(3 * NROWS, C))
    h3 = hout.reshape(3, NROWS, C)
    xnf = xn.reshape(NROWS, C)

    bng = jnp.stack([bn_g_0, bn_g_1, bn_g_2])[:, None, :]
    bnb = jnp.stack([bn_b_0, bn_b_1, bn_b_2])[:, None, :]
    attw = jnp.stack([att_w_0, att_w_1, att_w_2])
    attb = jnp.stack([att_b_0, att_b_1, att_b_2])[:, None, :]
    w1s = jnp.stack([mlpCA4_w1, mlpCA2_w1, mlpCA1_w1])
    b1s = jnp.stack([mlpCA4_b1, mlpCA2_b1, mlpCA1_b1])[:, None, :]
    w2s = jnp.stack([mlpCA4_w2, mlpCA2_w2, mlpCA1_w2])
    b2s = jnp.stack([mlpCA4_b2, mlpCA2_b2, mlpCA1_b2])[:, None, :]
    law = jnp.stack([lineA4_w, lineA2_w, lineA1_w])
    lab = jnp.stack([lineA4_b, lineA2_b, lineA1_b])[:, None, :]

    out = pl.pallas_call(
        _epi_body,
        out_shape=jax.ShapeDtypeStruct((B, C, N), jnp.float32),
    )(h3, xnf, bng, bnb, attw, attb, w1s, b1s, w2s, b2s, law, lab,
      lineFu_w, lineFu_b[None, :])
    return out


# speedup vs baseline: 73.0009x; 1.2906x over previous
import functools

import jax
import jax.numpy as jnp
from jax import lax
from jax.experimental import pallas as pl
from jax.experimental.pallas import tpu as pltpu
from jax.experimental.pallas import tpu_sc as plsc

C = 128
N = 2048
B = 2
BLK = 128
THR4 = 0.08
THR2 = 0.18
THR1 = 0.22
NROWS = B * N
LANES = 16
TJ = 32
NW = 32


def _prologue_body(xt_ref, gw_ref, gb_ref, xn_ref, adj_ref, utab_ref, vtab_ref):
    xt = xt_ref[0]
    s = jnp.sum(xt * xt, axis=1, keepdims=True)
    xn = xt / jnp.maximum(jnp.sqrt(s), 1e-12)
    xn_ref[0] = xn
    adj_ref[0] = lax.dot_general(xn, xn, (((1,), (1,)), ((), ())),
                                 preferred_element_type=jnp.float32)
    for l in range(3):
        w = gw_ref[l]
        w1 = w[:C]
        w2 = w[C:]
        u = jnp.dot(xn, w1 - w2, preferred_element_type=jnp.float32) + gb_ref[l]
        v = jnp.dot(xn, w2, preferred_element_type=jnp.float32)
        utab_ref[l, 0] = u
        vtab_ref[l, 0] = v


_SC_MESH = plsc.VectorSubcoreMesh(core_axis_name="c", subcore_axis_name="s")


@functools.partial(
    pl.kernel,
    out_type=jax.ShapeDtypeStruct((3 * NROWS, C), jnp.float32),
    mesh=_SC_MESH,
    compiler_params=pltpu.CompilerParams(needs_layout_passes=False),
    scratch_types=[
        pltpu.VMEM((BLK, C), jnp.float32),
        pltpu.VMEM((BLK, C), jnp.float32),
        pltpu.VMEM((BLK, C), jnp.float32),
        pltpu.VMEM((BLK, C), jnp.float32),
        pltpu.VMEM((BLK, C), jnp.float32),
        pltpu.VMEM((BLK, C), jnp.float32),
        pltpu.VMEM((TJ, BLK), jnp.float32),
        pltpu.VMEM((TJ, BLK), jnp.float32),
        pltpu.VMEM((TJ, C), jnp.float32),
        pltpu.VMEM((TJ, C), jnp.float32),
        pltpu.VMEM((TJ, C), jnp.float32),
        pltpu.VMEM((TJ, C), jnp.float32),
        pltpu.VMEM((BLK + LANES,), jnp.int32),
        pltpu.VMEM((BLK + LANES,), jnp.float32),
        pltpu.VMEM((BLK + LANES,), jnp.float32),
        pltpu.SemaphoreType.DMA,
        pltpu.SemaphoreType.DMA,
    ],
)
def _sc_core(adj_hbm, utab_hbm, vtab_hbm, hout_hbm,
             u_a, u_b, u2, h0, h1, h2, adjA, adjB, v1A, v1B, v2A, v2B,
             lbuf, gbuf, abuf, semA, semB):
    wid = lax.axis_index("s") * 2 + lax.axis_index("c")
    bbase = (wid // 16) * N
    i0 = (wid % 16) * BLK
    neg = jnp.full((LANES,), -jnp.inf, jnp.float32)
    bufsets = ((adjA, v1A, v2A, semA), (adjB, v1B, v2B, semB))

    pltpu.sync_copy(utab_hbm.at[pl.ds(2 * NROWS + bbase + i0, BLK)], u2)
    for href in (h0, h1, h2):
        def initrow(ri, _, href=href):
            for k in range(C // LANES):
                href[ri, pl.ds(k * LANES, LANES)] = neg
            return 0
        lax.fori_loop(0, BLK, initrow, 0)

    def sweep(thr, uref, href, vl_base, chunk0, nchunks, skip_own):
        def valid(ci):
            j0 = chunk0 + ci * TJ
            ok = ci < nchunks
            if skip_own:
                ok = ok & jnp.logical_not((j0 >= i0) & (j0 < i0 + BLK))
            return ok

        def copies(ci, bs):
            adjb, v1b, v2b, sem = bs
            j0 = chunk0 + ci * TJ
            return (
                pltpu.make_async_copy(
                    adj_hbm.at[pl.ds(bbase + j0, TJ), pl.ds(i0, BLK)],
                    adjb, sem),
                pltpu.make_async_copy(
                    vtab_hbm.at[pl.ds(vl_base + bbase + j0, TJ)], v1b, sem),
                pltpu.make_async_copy(
                    vtab_hbm.at[pl.ds(2 * NROWS + bbase + j0, TJ)], v2b, sem),
            )

        def start(ci, bs):
            @pl.when(valid(ci))
            def _():
                for cp in copies(ci, bs):
                    cp.start()

        def process(ci, bs):
            adjb, v1b, v2b, sem = bs

            @pl.when(valid(ci))
            def _():
                for cp in copies(ci, bs):
                    cp.wait()

                def per_j(jl, _):
                    avs = [adjb[jl, pl.ds(kk * LANES, LANES)]
                           for kk in range(BLK // LANES)]
                    ms = [a > thr for a in avs]
                    ns = [plsc.all_reduce_population_count(m)[0] for m in ms]
                    offs = []
                    cnt = jnp.int32(0)
                    for n in ns:
                        offs.append(cnt)
                        cnt = cnt + n

                    @pl.when(cnt > 0)
                    def _():
                        for kk in range(BLK // LANES):
                            @pl.when(ns[kk] > 0)
                            def _(kk=kk):
                                a = avs[kk]
                                m = ms[kk]
                                g = 1.0 / (1.0 + jnp.exp(-a))
                                ids = lax.iota(jnp.int32, LANES) + (kk * LANES)
                                plsc.store_compressed(
                                    lbuf.at[pl.ds(offs[kk], LANES)], ids,
                                    mask=m)
                                plsc.store_compressed(
                                    gbuf.at[pl.ds(offs[kk], LANES)], g, mask=m)
                                plsc.store_compressed(
                                    abuf.at[pl.ds(offs[kk], LANES)], a, mask=m)

                        v1r = [v1b[jl, pl.ds(k * LANES, LANES)]
                               for k in range(C // LANES)]
                        v2r = [v2b[jl, pl.ds(k * LANES, LANES)]
                               for k in range(C // LANES)]

                        @plsc.parallel_loop(0, cnt, 1, unroll=4)
                        def edge(e):
                            lane = lbuf[pl.ds(e, LANES)][0]
                            g = gbuf[pl.ds(e, LANES)][0]
                            av = abuf[pl.ds(e, LANES)][0]
                            for k in range(C // LANES):
                                sl = pl.ds(k * LANES, LANES)
                                cand = g * (uref[lane, sl] + v1r[k])
                                href[lane, sl] = jnp.maximum(href[lane, sl],
                                                             cand)

                            @pl.when(av > THR1)
                            def _():
                                for k in range(C // LANES):
                                    sl = pl.ds(k * LANES, LANES)
                                    cand2 = g * (u2[lane, sl] + v2r[k])
                                    h2[lane, sl] = jnp.maximum(h2[lane, sl],
                                                               cand2)
                    return 0

                lax.fori_loop(0, TJ, per_j, 0)

        start(0, bufsets[0])

        def pair(p, _):
            start(2 * p + 1, bufsets[1])
            process(2 * p, bufsets[0])
            start(2 * p + 2, bufsets[0])
            process(2 * p + 1, bufsets[1])
            return 0

        lax.fori_loop(0, nchunks // 2, pair, 0)

    pltpu.sync_copy(utab_hbm.at[pl.ds(bbase + i0, BLK)], u_a)
    sweep(THR4, u_a, h0, 0, i0, BLK // TJ, False)
    pltpu.sync_copy(utab_hbm.at[pl.ds(NROWS + bbase + i0, BLK)], u_b)
    sweep(THR2, u_b, h1, NROWS, 0, N // TJ, True)

    for l, href in ((0, h0), (1, h1), (2, h2)):
        def finrow(ri, _, href=href):
            for k in range(C // LANES):
                sl = pl.ds(k * LANES, LANES)
                v = href[ri, sl]
                href[ri, sl] = jnp.where(v > -1e37, v, 0.0)
            return 0
        lax.fori_loop(0, BLK, finrow, 0)
        pltpu.sync_copy(href, hout_hbm.at[pl.ds(l * NROWS + bbase + i0, BLK)])


def _sigmoid(z):
    return 1.0 / (1.0 + jnp.exp(-z))


def _epi_body(h_ref, xn_ref, bng_ref, bnb_ref, attw_ref, attb_ref,
              w1_ref, b1_ref, w2_ref, b2_ref, law_ref, lab_ref,
              fuw_ref, fub_ref, out_ref):
    xn = xn_ref[...]
    ys, ws, avals = [], [], []
    for l in range(3):
        h = h_ref[l]
        mu = jnp.mean(h, axis=0, keepdims=True)
        var = jnp.mean((h - mu) ** 2, axis=0, keepdims=True)
        hn = bng_ref[l] * (h - mu) / jnp.sqrt(var + 1e-5) + bnb_ref[l]
        hr = hn + xn
        gate = _sigmoid(jnp.dot(hr, attw_ref[l],
                                preferred_element_type=jnp.float32)
                        + attb_ref[l])
        y = jnp.maximum(gate * hr, 0.0)
        p0 = jnp.max(y[:N], axis=0, keepdims=True)
        p1 = jnp.max(y[N:], axis=0, keepdims=True)
        p = jnp.concatenate([p0, p1], axis=0)
        hid = jnp.maximum(jnp.dot(p, w1_ref[l],
                                  preferred_element_type=jnp.float32)
                          + b1_ref[l], 0.0)
        hcap = jnp.dot(hid, w2_ref[l],
                       preferred_element_type=jnp.float32) + b2_ref[l]
        w = _sigmoid(hcap)
        a = jnp.dot(w, law_ref[l],
                    preferred_element_type=jnp.float32) + lab_ref[l]
        ys.append(y)
        ws.append(w)
        avals.append(a)
    amax = jnp.maximum(jnp.maximum(avals[0], avals[1]), avals[2])
    es = [jnp.exp(a - amax) for a in avals]
    ssum = es[0] + es[1] + es[2]
    for b in range(B):
        fb = jnp.zeros((N, C), jnp.float32)
        for l in range(3):
            scale = (es[l][b:b + 1, :] / ssum[b:b + 1, :]) * ws[l][b:b + 1, :]
            fb = fb + scale * ys[l][b * N:(b + 1) * N]
        ob = jnp.dot(fb, fuw_ref[...],
                     preferred_element_type=jnp.float32) + fub_ref[...]
        out_ref[b] = ob.T


def kernel(x,
           gcn_w_0, gcn_b_0, bn_g_0, bn_b_0, att_w_0, att_b_0,
           gcn_w_1, gcn_b_1, bn_g_1, bn_b_1, att_w_1, att_b_1,
           gcn_w_2, gcn_b_2, bn_g_2, bn_b_2, att_w_2, att_b_2,
           lineA4_w, lineA4_b, mlpCA4_w1, mlpCA4_b1, mlpCA4_w2, mlpCA4_b2,
           lineA2_w, lineA2_b, mlpCA2_w1, mlpCA2_b1, mlpCA2_w2, mlpCA2_b2,
           lineA1_w, lineA1_b, mlpCA1_w1, mlpCA1_b1, mlpCA1_w2, mlpCA1_b2,
           lineFu_w, lineFu_b):
    xt = jnp.transpose(x, (0, 2, 1))
    gw = jnp.stack([gcn_w_0, gcn_w_1, gcn_w_2])
    gb = jnp.stack([gcn_b_0, gcn_b_1, gcn_b_2])[:, None, :]

    xn, adj, utab, vtab = pl.pallas_call(
        _prologue_body,
        grid=(B,),
        in_specs=[
            pl.BlockSpec((1, N, C), lambda b: (b, 0, 0)),
            pl.BlockSpec((3, 2 * C, C), lambda b: (0, 0, 0)),
            pl.BlockSpec((3, 1, C), lambda b: (0, 0, 0)),
        ],
        out_specs=[
            pl.BlockSpec((1, N, C), lambda b: (b, 0, 0)),
            pl.BlockSpec((1, N, N), lambda b: (b, 0, 0)),
            pl.BlockSpec((3, 1, N, C), lambda b: (0, b, 0, 0)),
            pl.BlockSpec((3, 1, N, C), lambda b: (0, b, 0, 0)),
        ],
        out_shape=[
            jax.ShapeDtypeStruct((B, N, C), jnp.float32),
            jax.ShapeDtypeStruct((B, N, N), jnp.float32),
            jax.ShapeDtypeStruct((3, B, N, C), jnp.float32),
            jax.ShapeDtypeStruct((3, B, N, C), jnp.float32),
        ],
    )(xt, gw, gb)

    hout = _sc_core(adj.reshape(NROWS, N), utab.reshape(3 * NROWS, C),
                    vtab.reshape(3 * NROWS, C))
    h3 = hout.reshape(3, NROWS, C)
    xnf = xn.reshape(NROWS, C)

    bng = jnp.stack([bn_g_0, bn_g_1, bn_g_2])[:, None, :]
    bnb = jnp.stack([bn_b_0, bn_b_1, bn_b_2])[:, None, :]
    attw = jnp.stack([att_w_0, att_w_1, att_w_2])
    attb = jnp.stack([att_b_0, att_b_1, att_b_2])[:, None, :]
    w1s = jnp.stack([mlpCA4_w1, mlpCA2_w1, mlpCA1_w1])
    b1s = jnp.stack([mlpCA4_b1, mlpCA2_b1, mlpCA1_b1])[:, None, :]
    w2s = jnp.stack([mlpCA4_w2, mlpCA2_w2, mlpCA1_w2])
    b2s = jnp.stack([mlpCA4_b2, mlpCA2_b2, mlpCA1_b2])[:, None, :]
    law = jnp.stack([lineA4_w, lineA2_w, lineA1_w])
    lab = jnp.stack([lineA4_b, lineA2_b, lineA1_b])[:, None, :]

    out = pl.pallas_call(
        _epi_body,
        out_shape=jax.ShapeDtypeStruct((B, C, N), jnp.float32),
    )(h3, xnf, bng, bnb, attw, attb, w1s, b1s, w2s, b2s, law, lab,
      lineFu_w, lineFu_b[None, :])
    return out
